# baseline (speedup 1.0000x reference)
import functools

import jax
import jax.numpy as jnp
from jax import lax
from jax.experimental import pallas as pl
from jax.experimental.pallas import tpu as pltpu
from jax.experimental.pallas import tpu_sc as plsc

_KSC = 32
_HB = 13

_NC, _NS, _L = 2, 16, 16
_OFFS = (0, 16, 32, 48, 64, 69)

_P5, _P4, _P3, _P2, _P1, _P0 = (-0.02368925, 0.1002872, -0.20866966,
                                0.32441181, -0.49918785, 0.99998187)

_DN = lax.GatherDimensionNumbers(offset_dims=(), collapsed_slice_dims=(0,),
                                 start_index_map=(0,))


def _splat(v, lane):
    idx = jnp.full((_L, 1), lane, jnp.int32)
    return lax.gather(v, idx, _DN, (1,),
                      mode=lax.GatherScatterMode.PROMISE_IN_BOUNDS)


def _bce(p, g):
    relu = jnp.maximum(p, 0.0)
    e = jnp.exp(-jnp.abs(p))
    q = _P5
    for cf in (_P4, _P3, _P2, _P1, _P0):
        q = q * e + cf
    return relu - p * g + q * e


def _make_sc(b_dim, h_dim, w_dim, a_dim, c_dim, ksc):
    rows = ksc * h_dim
    rpt = rows // (_NC * _NS)
    mesh = plsc.VectorSubcoreMesh(core_axis_name="c", subcore_axis_name="s")

    @functools.partial(
        pl.kernel,
        out_type=jax.ShapeDtypeStruct((_NC * _NS, _L), jnp.float32),
        mesh=mesh,
        scratch_types=[
            pltpu.VMEM((2, w_dim, a_dim, c_dim), jnp.float32),
            pltpu.VMEM((2, w_dim, a_dim, c_dim), jnp.float32),
            pltpu.VMEM((_L,), jnp.float32),
            pltpu.SemaphoreType.DMA((2, 2)),
        ],
    )
    def sc_loss(p_hbm, g_hbm, out, pbuf, gbuf, accv, sems):
        wid = lax.axis_index("s") * _NC + lax.axis_index("c")
        tiles_per_b = h_dim // rpt
        b = wid // tiles_per_b
        h0 = lax.rem(wid, tiles_per_b) * rpt

        lanei = lax.iota(jnp.int32, _L)
        w5 = jnp.where(lanei < 4, 5.0, 0.0)
        wc = jnp.where(lanei == 4, 1.0, 0.0)
        wr = jnp.where(lanei >= 5, 1.0, 0.0)
        wt = jnp.where(lanei >= 11, 1.0, 0.0)

        def start(r, s):
            pltpu.make_async_copy(p_hbm.at[b, h0 + r], pbuf.at[s],
                                  sems.at[s, 0]).start()
            pltpu.make_async_copy(g_hbm.at[b, h0 + r], gbuf.at[s],
                                  sems.at[s, 1]).start()

        def wait(s):
            pltpu.make_async_copy(p_hbm.at[0, 0], pbuf.at[s],
                                  sems.at[s, 0]).wait()
            pltpu.make_async_copy(g_hbm.at[0, 0], gbuf.at[s],
                                  sems.at[s, 1]).wait()

        def process(s, acc):
            def wbody(w, acc):
                for a in range(a_dim):
                    pv = [pbuf[s, w, a, pl.ds(o, _L)] for o in _OFFS]
                    gv = [gbuf[s, w, a, pl.ds(o, _L)] for o in _OFFS]
                    g4 = _splat(gv[0], 4)
                    d = gv[0] - pv[0]
                    acc = acc + w5 * (g4 * (d * d))
                    wb = wc * (0.5 + 0.5 * g4) + wr * g4
                    acc = acc + wb * _bce(pv[0], gv[0])
                    for k in range(1, 5):
                        acc = acc + g4 * _bce(pv[k], gv[k])
                    acc = acc + (wt * g4) * _bce(pv[5], gv[5])
                return acc

            return lax.fori_loop(0, w_dim, wbody, acc)

        start(0, 0)
        start(1, 1)
        acc = jnp.zeros((_L,), jnp.float32)

        def hbody(t, acc):
            r0 = 2 * t
            wait(0)
            acc = process(0, acc)

            @pl.when(r0 + 2 < rpt)
            def _():
                start(r0 + 2, 0)

            wait(1)
            acc = process(1, acc)

            @pl.when(r0 + 3 < rpt)
            def _():
                start(r0 + 3, 1)

            return acc

        acc = lax.fori_loop(0, rpt // 2, hbody, acc)
        accv[...] = acc
        pltpu.sync_copy(accv, out.at[wid])

    return sc_loss


def _make_tc(b_dim, h_dim, w_dim, a_dim, c_dim, ksc):
    hsteps = h_dim // _HB
    steps = (b_dim - ksc) * hsteps

    def body(p_hbm, g_hbm, out_ref, pbuf, gbuf, sem):
        i = pl.program_id(0)
        slot = lax.rem(i, 2)
        nxt = lax.rem(i + 1, 2)

        def start(step, slot_):
            b = ksc + step // hsteps
            h0 = lax.rem(step, hsteps) * _HB
            for a in range(a_dim):
                pltpu.make_async_copy(
                    p_hbm.at[b, pl.ds(h0, _HB), :, a, :],
                    pbuf.at[slot_, a], sem.at[slot_, 0, a]).start()
                pltpu.make_async_copy(
                    g_hbm.at[b, pl.ds(h0, _HB), :, a, :],
                    gbuf.at[slot_, a], sem.at[slot_, 1, a]).start()

        @pl.when(i == 0)
        def _prologue():
            start(i, slot)

        @pl.when(i + 1 < steps)
        def _prefetch():
            start(i + 1, nxt)

        for a in range(a_dim):
            pltpu.make_async_copy(
                p_hbm.at[0, pl.ds(0, _HB), :, a, :], pbuf.at[slot, a],
                sem.at[slot, 0, a]).wait()
            pltpu.make_async_copy(
                g_hbm.at[0, pl.ds(0, _HB), :, a, :], gbuf.at[slot, a],
                sem.at[slot, 1, a]).wait()

        s = jnp.float32(0.0)
        for a in range(a_dim):
            p = pbuf[slot, a]
            g = gbuf[slot, a]
            c = lax.broadcasted_iota(jnp.int32, p.shape, 2)
            g4 = g[..., 4:5]
            mse_w = jnp.where(c < 4, 5.0 * g4, 0.0)
            bce_w = jnp.where(c == 4, 0.5 + 0.5 * g4,
                              jnp.where(c >= 5, g4, 0.0))
            d = g - p
            bce = jnp.maximum(p, 0.0) - p * g + jnp.log1p(jnp.exp(-jnp.abs(p)))
            s = s + jnp.sum(mse_w * (d * d) + bce_w * bce)

        @pl.when(i == 0)
        def _init():
            out_ref[0, 0] = s

        @pl.when(i != 0)
        def _acc():
            out_ref[0, 0] = out_ref[0, 0] + s

    def run(pred, gt):
        return pl.pallas_call(
            body,
            grid=(steps,),
            in_specs=[
                pl.BlockSpec(memory_space=pl.ANY),
                pl.BlockSpec(memory_space=pl.ANY),
            ],
            out_specs=pl.BlockSpec((1, 1), lambda i: (0, 0),
                                   memory_space=pltpu.SMEM),
            out_shape=jax.ShapeDtypeStruct((1, 1), jnp.float32),
            scratch_shapes=[
                pltpu.VMEM((2, a_dim, _HB, w_dim, c_dim), jnp.float32),
                pltpu.VMEM((2, a_dim, _HB, w_dim, c_dim), jnp.float32),
                pltpu.SemaphoreType.DMA((2, 2, a_dim)),
            ],
        )(pred, gt)

    return run


def kernel(pred, gt):
    b_dim, h_dim, w_dim, a_dim, c_dim = pred.shape
    total = jnp.float32(0.0)
    if _KSC > 0:
        sc = _make_sc(b_dim, h_dim, w_dim, a_dim, c_dim, _KSC)
        total = total + jnp.sum(sc(pred, gt))
    if _KSC < b_dim:
        tc = _make_tc(b_dim, h_dim, w_dim, a_dim, c_dim, _KSC)
        total = total + tc(pred, gt)[0, 0]
    return total * (1.0 / b_dim)


# speedup vs baseline: 1.3012x; 1.3012x over previous
import functools

import jax
import jax.numpy as jnp
from jax import lax
from jax.experimental import pallas as pl
from jax.experimental.pallas import tpu as pltpu
from jax.experimental.pallas import tpu_sc as plsc

_KSC = 16
_HB = 13

_NC, _NS, _L = 2, 16, 16
_OFFS = (0, 16, 32, 48, 64, 69)

_P5, _P4, _P3, _P2, _P1, _P0 = (-0.02368925, 0.1002872, -0.20866966,
                                0.32441181, -0.49918785, 0.99998187)

_DN = lax.GatherDimensionNumbers(offset_dims=(), collapsed_slice_dims=(0,),
                                 start_index_map=(0,))


def _splat(v, lane):
    idx = jnp.full((_L, 1), lane, jnp.int32)
    return lax.gather(v, idx, _DN, (1,),
                      mode=lax.GatherScatterMode.PROMISE_IN_BOUNDS)


def _bce(p, g):
    relu = jnp.maximum(p, 0.0)
    e = jnp.exp(-jnp.abs(p))
    q = _P5
    for cf in (_P4, _P3, _P2, _P1, _P0):
        q = q * e + cf
    return relu - p * g + q * e


def _make_sc(b_dim, h_dim, w_dim, a_dim, c_dim, ksc):
    rows = ksc * h_dim
    rpt = rows // (_NC * _NS)
    mesh = plsc.VectorSubcoreMesh(core_axis_name="c", subcore_axis_name="s")

    @functools.partial(
        pl.kernel,
        out_type=jax.ShapeDtypeStruct((_NC * _NS, _L), jnp.float32),
        mesh=mesh,
        scratch_types=[
            pltpu.VMEM((2, w_dim, a_dim, c_dim), jnp.float32),
            pltpu.VMEM((2, w_dim, a_dim, c_dim), jnp.float32),
            pltpu.VMEM((_L,), jnp.float32),
            pltpu.SemaphoreType.DMA((2, 2)),
        ],
    )
    def sc_loss(p_hbm, g_hbm, out, pbuf, gbuf, accv, sems):
        wid = lax.axis_index("s") * _NC + lax.axis_index("c")
        tiles_per_b = h_dim // rpt
        b = wid // tiles_per_b
        h0 = lax.rem(wid, tiles_per_b) * rpt

        lanei = lax.iota(jnp.int32, _L)
        w5 = jnp.where(lanei < 4, 5.0, 0.0)
        wc = jnp.where(lanei == 4, 1.0, 0.0)
        wr = jnp.where(lanei >= 5, 1.0, 0.0)
        wt = jnp.where(lanei >= 11, 1.0, 0.0)

        def start(r, s):
            pltpu.make_async_copy(p_hbm.at[b, h0 + r], pbuf.at[s],
                                  sems.at[s, 0]).start()
            pltpu.make_async_copy(g_hbm.at[b, h0 + r], gbuf.at[s],
                                  sems.at[s, 1]).start()

        def wait(s):
            pltpu.make_async_copy(p_hbm.at[0, 0], pbuf.at[s],
                                  sems.at[s, 0]).wait()
            pltpu.make_async_copy(g_hbm.at[0, 0], gbuf.at[s],
                                  sems.at[s, 1]).wait()

        def process(s, acc):
            def wbody(w, acc):
                for a in range(a_dim):
                    pv = [pbuf[s, w, a, pl.ds(o, _L)] for o in _OFFS]
                    gv = [gbuf[s, w, a, pl.ds(o, _L)] for o in _OFFS]
                    g4 = _splat(gv[0], 4)
                    d = gv[0] - pv[0]
                    acc = acc + w5 * (g4 * (d * d))
                    wb = wc * (0.5 + 0.5 * g4) + wr * g4
                    acc = acc + wb * _bce(pv[0], gv[0])
                    for k in range(1, 5):
                        acc = acc + g4 * _bce(pv[k], gv[k])
                    acc = acc + (wt * g4) * _bce(pv[5], gv[5])
                return acc

            return lax.fori_loop(0, w_dim, wbody, acc)

        start(0, 0)
        start(1, 1)
        acc = jnp.zeros((_L,), jnp.float32)

        def hbody(t, acc):
            r0 = 2 * t
            wait(0)
            acc = process(0, acc)

            @pl.when(r0 + 2 < rpt)
            def _():
                start(r0 + 2, 0)

            wait(1)
            acc = process(1, acc)

            @pl.when(r0 + 3 < rpt)
            def _():
                start(r0 + 3, 1)

            return acc

        acc = lax.fori_loop(0, rpt // 2, hbody, acc)
        accv[...] = acc
        pltpu.sync_copy(accv, out.at[wid])

    return sc_loss


def _make_tc(b_dim, h_dim, w_dim, a_dim, c_dim, ksc):
    hsteps = h_dim // _HB
    steps = (b_dim - ksc) * hsteps

    def body(p_hbm, g_hbm, out_ref, pbuf, gbuf, sem):
        i = pl.program_id(0)
        slot = lax.rem(i, 2)
        nxt = lax.rem(i + 1, 2)

        def start(step, slot_):
            b = ksc + step // hsteps
            h0 = lax.rem(step, hsteps) * _HB
            for a in range(a_dim):
                pltpu.make_async_copy(
                    p_hbm.at[b, pl.ds(h0, _HB), :, a, :],
                    pbuf.at[slot_, a], sem.at[slot_, 0, a]).start()
                pltpu.make_async_copy(
                    g_hbm.at[b, pl.ds(h0, _HB), :, a, :],
                    gbuf.at[slot_, a], sem.at[slot_, 1, a]).start()

        @pl.when(i == 0)
        def _prologue():
            start(i, slot)

        @pl.when(i + 1 < steps)
        def _prefetch():
            start(i + 1, nxt)

        for a in range(a_dim):
            pltpu.make_async_copy(
                p_hbm.at[0, pl.ds(0, _HB), :, a, :], pbuf.at[slot, a],
                sem.at[slot, 0, a]).wait()
            pltpu.make_async_copy(
                g_hbm.at[0, pl.ds(0, _HB), :, a, :], gbuf.at[slot, a],
                sem.at[slot, 1, a]).wait()

        s = jnp.float32(0.0)
        for a in range(a_dim):
            p = pbuf[slot, a]
            g = gbuf[slot, a]
            c = lax.broadcasted_iota(jnp.int32, p.shape, 2)
            g4 = g[..., 4:5]
            mse_w = jnp.where(c < 4, 5.0 * g4, 0.0)
            bce_w = jnp.where(c == 4, 0.5 + 0.5 * g4,
                              jnp.where(c >= 5, g4, 0.0))
            d = g - p
            bce = jnp.maximum(p, 0.0) - p * g + jnp.log1p(jnp.exp(-jnp.abs(p)))
            s = s + jnp.sum(mse_w * (d * d) + bce_w * bce)

        @pl.when(i == 0)
        def _init():
            out_ref[0, 0] = s

        @pl.when(i != 0)
        def _acc():
            out_ref[0, 0] = out_ref[0, 0] + s

    def run(pred, gt):
        return pl.pallas_call(
            body,
            grid=(steps,),
            in_specs=[
                pl.BlockSpec(memory_space=pl.ANY),
                pl.BlockSpec(memory_space=pl.ANY),
            ],
            out_specs=pl.BlockSpec((1, 1), lambda i: (0, 0),
                                   memory_space=pltpu.SMEM),
            out_shape=jax.ShapeDtypeStruct((1, 1), jnp.float32),
            scratch_shapes=[
                pltpu.VMEM((2, a_dim, _HB, w_dim, c_dim), jnp.float32),
                pltpu.VMEM((2, a_dim, _HB, w_dim, c_dim), jnp.float32),
                pltpu.SemaphoreType.DMA((2, 2, a_dim)),
            ],
        )(pred, gt)

    return run


def kernel(pred, gt):
    b_dim, h_dim, w_dim, a_dim, c_dim = pred.shape
    total = jnp.float32(0.0)
    if _KSC > 0:
        sc = _make_sc(b_dim, h_dim, w_dim, a_dim, c_dim, _KSC)
        total = total + jnp.sum(sc(pred, gt))
    if _KSC < b_dim:
        tc = _make_tc(b_dim, h_dim, w_dim, a_dim, c_dim, _KSC)
        total = total + tc(pred, gt)[0, 0]
    return total * (1.0 / b_dim)
